# baseline (speedup 1.0000x reference)
import jax
import jax.numpy as jnp
from jax import lax
from jax.experimental import pallas as pl
from jax.experimental.pallas import tpu as pltpu
from jax.experimental.pallas import tpu_sc as plsc

B = 16
S = 4096
D = 512
T = 32768
NC = 2
NS = 16
NW = NC * NS
HALF = S // 2


def _sc_copy_body(flat_hbm, lo_hbm, hi_hbm, zeros_hbm, out_hbm, lo_v, hi_v):
    wid = lax.axis_index("s") * NC + lax.axis_index("c")
    b = wid // 2
    h = wid % 2
    pltpu.sync_copy(lo_hbm, lo_v.at[pl.ds(0, B)])
    pltpu.sync_copy(hi_hbm, hi_v.at[pl.ds(0, B)])
    start = lo_v[pl.ds(b, 16)][0]
    end = hi_v[pl.ds(b, 16)][0]
    seg_len = jnp.minimum(end - start, S)
    cnt = jnp.clip(seg_len - h * HALF, 0, HALF)
    pad = HALF - cnt
    src0 = start + h * HALF
    dst0 = b * S + h * HALF

    for k in range(11, -1, -1):
        sz = 1 << k
        off = cnt & ~((sz << 1) - 1)

        @pl.when((cnt & sz) != 0)
        def _():
            pltpu.sync_copy(flat_hbm.at[pl.ds((src0 + off) * D, sz * D)],
                            out_hbm.at[pl.ds((dst0 + off) * D, sz * D)])

        zoff = pad & ~((sz << 1) - 1)

        @pl.when((pad & sz) != 0)
        def _():
            pltpu.sync_copy(
                zeros_hbm.at[pl.ds(0, sz * D)],
                out_hbm.at[pl.ds((dst0 + cnt + zoff) * D, sz * D)])


_sc_copy = pl.kernel(
    _sc_copy_body,
    out_type=jax.ShapeDtypeStruct((B * S * D,), jnp.float32),
    mesh=plsc.VectorSubcoreMesh(core_axis_name="c", subcore_axis_name="s"),
    scratch_types=[
        pltpu.VMEM((2 * B,), jnp.int32),
        pltpu.VMEM((2 * B,), jnp.int32),
    ],
)


def _mask_body(cu_ref, mask_ref):
    iota = lax.broadcasted_iota(jnp.int32, (1, S), 1)
    for b in range(B):
        l = jnp.minimum(cu_ref[b + 1] - cu_ref[b], S)
        mask_ref[b:b + 1, :] = iota < l


def _mask_call(cu):
    return pl.pallas_call(
        _mask_body,
        in_specs=[pl.BlockSpec(memory_space=pltpu.SMEM)],
        out_shape=jax.ShapeDtypeStruct((B, S), jnp.bool_),
    )(cu)


def kernel(flat_encodings, cu_seqlens):
    cu = cu_seqlens.astype(jnp.int32)
    lo = cu[:B]
    hi = cu[1:B + 1]
    zeros = jnp.zeros((HALF * D,), jnp.float32)
    out = _sc_copy(flat_encodings.reshape(T * D), lo, hi, zeros)
    mask = _mask_call(cu)
    return out.reshape(B, S, D), mask


# speedup vs baseline: 13.5653x; 13.5653x over previous
import jax
import jax.numpy as jnp
from jax import lax
from jax.experimental import pallas as pl
from jax.experimental.pallas import tpu as pltpu
from jax.experimental.pallas import tpu_sc as plsc

B = 16
S = 4096
D = 512
T = 32768
NC = 2
NS = 16
NW = NC * NS
HALF = S // 2
C = 64
CD = C * D
NCHUNK = HALF // C


def _sc_copy_body(flat_hbm, lo_hbm, hi_hbm, zeros_hbm, out_hbm,
                  lo_v, hi_v, zbuf, buf0, buf1,
                  gsem0, gsem1, ssem0, ssem1, zsem):
    wid = lax.axis_index("s") * NC + lax.axis_index("c")
    b = wid // 2
    h = wid % 2
    pltpu.sync_copy(lo_hbm, lo_v.at[pl.ds(0, B)])
    pltpu.sync_copy(hi_hbm, hi_v.at[pl.ds(0, B)])
    pltpu.sync_copy(zeros_hbm, zbuf)
    start = lo_v[pl.ds(b, 16)][0]
    end = hi_v[pl.ds(b, 16)][0]
    seg_len = jnp.minimum(end - start, S)
    cnt = jnp.clip(seg_len - h * HALF, 0, HALF)
    src0 = start + h * HALF
    dst0 = b * S + h * HALF
    nfull = cnt // C
    r = cnt - nfull * C

    bufs = (buf0, buf1)
    gsems = (gsem0, gsem1)
    ssems = (ssem0, ssem1)

    def gather(c, p):
        return pltpu.make_async_copy(
            flat_hbm.at[pl.ds((src0 + c * C) * D, CD)], bufs[p], gsems[p])

    def scatter(c, p):
        return pltpu.make_async_copy(
            bufs[p], out_hbm.at[pl.ds((dst0 + c * C) * D, CD)], ssems[p])

    def zscatter(c):
        return pltpu.make_async_copy(
            zbuf, out_hbm.at[pl.ds((dst0 + c * C) * D, CD)], zsem)

    for c in range(NCHUNK):
        @pl.when(c >= nfull)
        def _(c=c):
            zscatter(c).start()

    for c0 in range(0, NCHUNK, 2):
        for p in range(2):
            c = c0 + p

            @pl.when(c < nfull)
            def _(c=c, p=p):
                if c >= 2:
                    scatter(c - 2, p).wait()
                gather(c, p).start()
        for p in range(2):
            c = c0 + p

            @pl.when(c < nfull)
            def _(c=c, p=p):
                gather(c, p).wait()
                scatter(c, p).start()

    @pl.when(nfull >= 1)
    def _():
        scatter(0, 0).wait()

    @pl.when(nfull >= 2)
    def _():
        scatter(0, 1).wait()

    for c in range(NCHUNK):
        @pl.when(c >= nfull)
        def _(c=c):
            zscatter(c).wait()

    @pl.when((r > 0) & (cnt >= C))
    def _():
        o = cnt - C
        pltpu.make_async_copy(
            flat_hbm.at[pl.ds((src0 + o) * D, CD)], buf0, gsem0).start()
        pltpu.make_async_copy(
            flat_hbm.at[pl.ds((src0 + o) * D, CD)], buf0, gsem0).wait()
        pltpu.make_async_copy(
            buf0, out_hbm.at[pl.ds((dst0 + o) * D, CD)], ssem0).start()
        pltpu.make_async_copy(
            buf0, out_hbm.at[pl.ds((dst0 + o) * D, CD)], ssem0).wait()

    @pl.when((r > 0) & (cnt < C))
    def _():
        for k in range(5, -1, -1):
            sz = 1 << k
            off = r & ~((sz << 1) - 1)

            @pl.when((r & sz) != 0)
            def _(sz=sz, off=off):
                pltpu.sync_copy(
                    flat_hbm.at[pl.ds((src0 + off) * D, sz * D)],
                    out_hbm.at[pl.ds((dst0 + off) * D, sz * D)])


_sc_copy = pl.kernel(
    _sc_copy_body,
    out_type=jax.ShapeDtypeStruct((B * S * D,), jnp.float32),
    mesh=plsc.VectorSubcoreMesh(core_axis_name="c", subcore_axis_name="s"),
    scratch_types=[
        pltpu.VMEM((2 * B,), jnp.int32),
        pltpu.VMEM((2 * B,), jnp.int32),
        pltpu.VMEM((CD,), jnp.float32),
        pltpu.VMEM((CD,), jnp.float32),
        pltpu.VMEM((CD,), jnp.float32),
        pltpu.SemaphoreType.DMA,
        pltpu.SemaphoreType.DMA,
        pltpu.SemaphoreType.DMA,
        pltpu.SemaphoreType.DMA,
        pltpu.SemaphoreType.DMA,
    ],
)


def _mask_body(cu_ref, mask_ref):
    iota = lax.broadcasted_iota(jnp.int32, (1, S), 1)
    for b in range(B):
        l = jnp.minimum(cu_ref[b + 1] - cu_ref[b], S)
        mask_ref[b:b + 1, :] = iota < l


def _mask_call(cu):
    return pl.pallas_call(
        _mask_body,
        in_specs=[pl.BlockSpec(memory_space=pltpu.SMEM)],
        out_shape=jax.ShapeDtypeStruct((B, S), jnp.bool_),
    )(cu)


def kernel(flat_encodings, cu_seqlens):
    cu = cu_seqlens.astype(jnp.int32)
    lo = cu[:B]
    hi = cu[1:B + 1]
    zeros = jnp.zeros((CD,), jnp.float32)
    out = _sc_copy(flat_encodings.reshape(T * D), lo, hi, zeros)
    mask = _mask_call(cu)
    return out.reshape(B, S, D), mask


# speedup vs baseline: 32.7244x; 2.4124x over previous
import jax
import jax.numpy as jnp
from jax import lax
from jax.experimental import pallas as pl
from jax.experimental.pallas import tpu as pltpu
from jax.experimental.pallas import tpu_sc as plsc

B = 16
S = 4096
D = 512
T = 32768
NC = 2
NS = 16
NW = NC * NS
HALF = S // 2
C = 64
NCHUNK = HALF // C


def _sc_copy_body(flat_hbm, lo_hbm, hi_hbm, zeros_hbm, out_hbm,
                  lo_v, hi_v, idx_v, zbuf, buf0, buf1,
                  gsem0, gsem1, ssem0, ssem1, zsem):
    wid = lax.axis_index("s") * NC + lax.axis_index("c")
    b = wid // 2
    h = wid % 2
    pltpu.sync_copy(lo_hbm, lo_v.at[pl.ds(0, B)])
    pltpu.sync_copy(hi_hbm, hi_v.at[pl.ds(0, B)])
    pltpu.sync_copy(zeros_hbm, zbuf)
    start = lo_v[pl.ds(b, 16)][0]
    end = hi_v[pl.ds(b, 16)][0]
    seg_len = jnp.minimum(end - start, S)
    cnt = jnp.clip(seg_len - h * HALF, 0, HALF)
    src0 = start + h * HALF
    dst0 = b * S + h * HALF
    nfull = cnt // C
    rem = cnt - nfull * C
    zstart = nfull + (rem > 0).astype(jnp.int32)

    iota = lax.iota(jnp.int32, 16)

    def _bld(j, carry):
        idx_v[pl.ds(j * 16, 16)] = jnp.minimum(src0 + j * 16 + iota, T - 1)
        return carry

    lax.fori_loop(0, HALF // 16, _bld, 0)

    bufs = (buf0, buf1)
    gsems = (gsem0, gsem1)
    ssems = (ssem0, ssem1)

    def gather(c, p):
        return pltpu.make_async_copy(
            flat_hbm.at[idx_v.at[pl.ds(c * C, C)]], bufs[p], gsems[p])

    def scatter(c, p):
        return pltpu.make_async_copy(
            bufs[p], out_hbm.at[pl.ds(dst0 + c * C, C)], ssems[p])

    def zscatter(c):
        return pltpu.make_async_copy(
            zbuf, out_hbm.at[pl.ds(dst0 + c * C, C)], zsem)

    for c in range(NCHUNK):
        @pl.when(c >= zstart)
        def _(c=c):
            zscatter(c).start()

    for c0 in range(0, NCHUNK, 2):
        for p in range(2):
            c = c0 + p

            @pl.when(c < nfull)
            def _(c=c, p=p):
                if c >= 2:
                    scatter(c - 2, p).wait()
                gather(c, p).start()
        for p in range(2):
            c = c0 + p

            @pl.when(c < nfull)
            def _(c=c, p=p):
                gather(c, p).wait()
                scatter(c, p).start()

    @pl.when(nfull >= 1)
    def _():
        scatter(0, 0).wait()

    @pl.when(nfull >= 2)
    def _():
        scatter(0, 1).wait()

    @pl.when(rem > 0)
    def _():
        gather(nfull, 0).start()
        gather(nfull, 0).wait()
        zero16 = jnp.zeros((16,), jnp.float32)

        def _zr(rw, carry):
            for k in range(D // 16):
                buf0[rw, pl.ds(k * 16, 16)] = zero16
            return carry

        lax.fori_loop(rem, C, _zr, 0)
        scatter(nfull, 0).start()
        scatter(nfull, 0).wait()

    for c in range(NCHUNK):
        @pl.when(c >= zstart)
        def _(c=c):
            zscatter(c).wait()


_sc_copy = pl.kernel(
    _sc_copy_body,
    out_type=jax.ShapeDtypeStruct((B * S, D), jnp.float32),
    mesh=plsc.VectorSubcoreMesh(core_axis_name="c", subcore_axis_name="s"),
    scratch_types=[
        pltpu.VMEM((2 * B,), jnp.int32),
        pltpu.VMEM((2 * B,), jnp.int32),
        pltpu.VMEM((HALF,), jnp.int32),
        pltpu.VMEM((C, D), jnp.float32),
        pltpu.VMEM((C, D), jnp.float32),
        pltpu.VMEM((C, D), jnp.float32),
        pltpu.SemaphoreType.DMA,
        pltpu.SemaphoreType.DMA,
        pltpu.SemaphoreType.DMA,
        pltpu.SemaphoreType.DMA,
        pltpu.SemaphoreType.DMA,
    ],
)


def _mask_body(cu_ref, mask_ref):
    iota = lax.broadcasted_iota(jnp.int32, (1, S), 1)
    for b in range(B):
        l = jnp.minimum(cu_ref[b + 1] - cu_ref[b], S)
        mask_ref[b:b + 1, :] = iota < l


def _mask_call(cu):
    return pl.pallas_call(
        _mask_body,
        in_specs=[pl.BlockSpec(memory_space=pltpu.SMEM)],
        out_shape=jax.ShapeDtypeStruct((B, S), jnp.bool_),
    )(cu)


def kernel(flat_encodings, cu_seqlens):
    cu = cu_seqlens.astype(jnp.int32)
    lo = cu[:B]
    hi = cu[1:B + 1]
    zeros = jnp.zeros((C, D), jnp.float32)
    out = _sc_copy(flat_encodings, lo, hi, zeros)
    mask = _mask_call(cu)
    return out.reshape(B, S, D), mask


# speedup vs baseline: 37.5911x; 1.1487x over previous
import jax
import jax.numpy as jnp
from jax import lax
from jax.experimental import pallas as pl
from jax.experimental.pallas import tpu as pltpu
from jax.experimental.pallas import tpu_sc as plsc

B = 16
S = 4096
D = 512
T = 32768
NC = 2
NS = 16
NW = NC * NS
C = 64
JPE = S // C
NT = 2 * B


def _sc_copy_body(flat_hbm, lo_hbm, hi_hbm, zeros_hbm, out_hbm,
                  lo_v, hi_v, idx0, idx1, zbuf, buf0, buf1,
                  gsem0, gsem1, ssem0, ssem1, zsem):
    w = lax.axis_index("s") * NC + lax.axis_index("c")
    pltpu.sync_copy(lo_hbm, lo_v.at[pl.ds(0, B)])
    pltpu.sync_copy(hi_hbm, hi_v.at[pl.ds(0, B)])
    pltpu.sync_copy(zeros_hbm, zbuf)
    iota = lax.iota(jnp.int32, 16)

    bufs = (buf0, buf1)
    idxs = (idx0, idx1)
    gsems = (gsem0, gsem1)
    ssems = (ssem0, ssem1)

    js = []
    dsts = []
    srcs = []
    vs = []
    for t in range(NT):
        b = t // 2
        j = ((w + 2 * b) & 31) + 32 * (t & 1)
        lo_b = lo_v[pl.ds(b, 16)][0]
        eff_b = jnp.minimum(hi_v[pl.ds(b, 16)][0] - lo_b, S)
        js.append(j)
        dsts.append(b * S + j * C)
        srcs.append(lo_b + j * C)
        vs.append(jnp.clip(eff_b - j * C, 0, C))

    def gather(t, p):
        return pltpu.make_async_copy(
            flat_hbm.at[idxs[p]], bufs[p], gsems[p])

    def scatter(t, p):
        return pltpu.make_async_copy(
            bufs[p], out_hbm.at[pl.ds(dsts[t], C)], ssems[p])

    def zscatter(t):
        return pltpu.make_async_copy(
            zbuf, out_hbm.at[pl.ds(dsts[t], C)], zsem)

    for t in range(NT):
        @pl.when(vs[t] == 0)
        def _(t=t):
            zscatter(t).start()

    zero16 = jnp.zeros((16,), jnp.float32)
    pending = [None] * 2
    for t0 in range(0, NT, 2):
        for p in range(2):
            t = t0 + p

            @pl.when(vs[t] > 0)
            def _(t=t, p=p):
                if pending[p] is not None:
                    @pl.when(pending[p])
                    def _():
                        scatter(t, p).wait()
                for k in range(C // 16):
                    idxs[p][pl.ds(k * 16, 16)] = jnp.minimum(
                        srcs[t] + k * 16 + iota, T - 1)
                gather(t, p).start()
        for p in range(2):
            t = t0 + p

            @pl.when(vs[t] > 0)
            def _(t=t, p=p):
                gather(t, p).wait()

                @pl.when(vs[t] < C)
                def _():
                    def _zr(rw, carry):
                        for k in range(D // 16):
                            bufs[p][rw, pl.ds(k * 16, 16)] = zero16
                        return carry

                    lax.fori_loop(vs[t], C, _zr, 0)

                scatter(t, p).start()
            cond = vs[t] > 0
            pending[p] = cond if pending[p] is None else (pending[p] | cond)

    for p in range(2):
        if pending[p] is not None:
            @pl.when(pending[p])
            def _(p=p):
                scatter(0, p).wait()

    for t in range(NT):
        @pl.when(vs[t] == 0)
        def _(t=t):
            zscatter(t).wait()


_SCRATCH = [
    pltpu.VMEM((2 * B,), jnp.int32),
    pltpu.VMEM((2 * B,), jnp.int32),
    pltpu.VMEM((C,), jnp.int32),
    pltpu.VMEM((C,), jnp.int32),
    pltpu.VMEM((C, D), jnp.float32),
    pltpu.VMEM((C, D), jnp.float32),
    pltpu.VMEM((C, D), jnp.float32),
    pltpu.SemaphoreType.DMA,
    pltpu.SemaphoreType.DMA,
    pltpu.SemaphoreType.DMA,
    pltpu.SemaphoreType.DMA,
    pltpu.SemaphoreType.DMA,
]

_sc_copy = pl.kernel(
    _sc_copy_body,
    out_type=jax.ShapeDtypeStruct((B * S, D), jnp.float32),
    mesh=plsc.VectorSubcoreMesh(core_axis_name="c", subcore_axis_name="s"),
    scratch_types=_SCRATCH,
)


def _mask_body(cu_ref, mask_ref):
    iota = lax.broadcasted_iota(jnp.int32, (1, S), 1)
    for b in range(B):
        l = jnp.minimum(cu_ref[b + 1] - cu_ref[b], S)
        mask_ref[b:b + 1, :] = iota < l


def _mask_call(cu):
    return pl.pallas_call(
        _mask_body,
        in_specs=[pl.BlockSpec(memory_space=pltpu.SMEM)],
        out_shape=jax.ShapeDtypeStruct((B, S), jnp.bool_),
    )(cu)


def kernel(flat_encodings, cu_seqlens):
    cu = cu_seqlens.astype(jnp.int32)
    lo = cu[:B]
    hi = cu[1:B + 1]
    zeros = jnp.zeros((C, D), jnp.float32)
    out = _sc_copy(flat_encodings, lo, hi, zeros)
    mask = _mask_call(cu)
    return out.reshape(B, S, D), mask


# speedup vs baseline: 44.4643x; 1.1828x over previous
import jax
import jax.numpy as jnp
from jax import lax
from jax.experimental import pallas as pl
from jax.experimental.pallas import tpu as pltpu
from jax.experimental.pallas import tpu_sc as plsc

B = 16
S = 4096
D = 512
T = 32768
NC = 2
NS = 16
NW = NC * NS
C = 64
JPE = S // C


def _sc_copy_body(flat_hbm, cu_hbm, out_hbm,
                  cu_v, idx0, idx1, zbuf, buf0, buf1,
                  gsem0, gsem1, ssem0, ssem1, zsem):
    w = lax.axis_index("s") * NC + lax.axis_index("c")
    pltpu.sync_copy(cu_hbm, cu_v.at[pl.ds(0, 2 * B)])
    iota = lax.iota(jnp.int32, 16)
    zero16 = jnp.zeros((16,), jnp.float32)

    def _zb(rw, carry):
        for k in range(D // 16):
            zbuf[rw, pl.ds(k * 16, 16)] = zero16
        return carry

    lax.fori_loop(0, C, _zb, 0)

    bufs = (buf0, buf1)
    idxs = (idx0, idx1)
    gsems = (gsem0, gsem1)
    ssems = (ssem0, ssem1)

    def chunk_params(b, p):
        j = ((w + 2 * b) & 31) + 32 * p
        lo_b = cu_v[pl.ds(b, 16)][0]
        eff_b = jnp.minimum(cu_v[pl.ds(b + B, 16)][0] - lo_b, S)
        dst = b * S + j * C
        src = lo_b + j * C
        v = jnp.clip(eff_b - j * C, 0, C)
        return src, dst, v

    def gather(p):
        return pltpu.make_async_copy(flat_hbm.at[idxs[p]], bufs[p], gsems[p])

    def scatter(dst, p):
        return pltpu.make_async_copy(
            bufs[p], out_hbm.at[pl.ds(dst, C)], ssems[p])

    def zscatter(dst):
        return pltpu.make_async_copy(zbuf, out_hbm.at[pl.ds(dst, C)], zsem)

    def _zs(b, nz):
        for p in range(2):
            src, dst, v = chunk_params(b, p)

            @pl.when(v == 0)
            def _(dst=dst):
                zscatter(dst).start()
            nz = nz + (v == 0).astype(jnp.int32)
        return nz

    nzero = lax.fori_loop(0, B, _zs, jnp.int32(0))

    def _ring(b, pending):
        prms = [chunk_params(b, p) for p in range(2)]
        for p in range(2):
            src, dst, v = prms[p]

            @pl.when(v > 0)
            def _(src=src, p=p):
                @pl.when(pending[p])
                def _():
                    scatter(0, p).wait()
                for k in range(C // 16):
                    idxs[p][pl.ds(k * 16, 16)] = jnp.minimum(
                        src + k * 16 + iota, T - 1)
                gather(p).start()
        new_pending = []
        for p in range(2):
            src, dst, v = prms[p]

            @pl.when(v > 0)
            def _(dst=dst, v=v, p=p):
                gather(p).wait()

                @pl.when(v < C)
                def _():
                    def _zr(rw, carry):
                        for k in range(D // 16):
                            bufs[p][rw, pl.ds(k * 16, 16)] = zero16
                        return carry

                    lax.fori_loop(v, C, _zr, 0)

                scatter(dst, p).start()
            new_pending.append(pending[p] | (v > 0))
        return tuple(new_pending)

    pending = lax.fori_loop(
        0, B, _ring, (jnp.bool_(False), jnp.bool_(False)))

    for p in range(2):
        @pl.when(pending[p])
        def _(p=p):
            scatter(0, p).wait()

    def _zd(i, carry):
        zscatter(0).wait()
        return carry

    lax.fori_loop(0, nzero, _zd, 0)


_SCRATCH = [
    pltpu.VMEM((3 * B,), jnp.int32),
    pltpu.VMEM((C,), jnp.int32),
    pltpu.VMEM((C,), jnp.int32),
    pltpu.VMEM((C, D), jnp.float32),
    pltpu.VMEM((C, D), jnp.float32),
    pltpu.VMEM((C, D), jnp.float32),
    pltpu.SemaphoreType.DMA,
    pltpu.SemaphoreType.DMA,
    pltpu.SemaphoreType.DMA,
    pltpu.SemaphoreType.DMA,
    pltpu.SemaphoreType.DMA,
]

_sc_copy = pl.kernel(
    _sc_copy_body,
    out_type=jax.ShapeDtypeStruct((B * S, D), jnp.float32),
    mesh=plsc.VectorSubcoreMesh(core_axis_name="c", subcore_axis_name="s"),
    scratch_types=_SCRATCH,
)


def _mask_body(cu_ref, mask_ref):
    iota = lax.broadcasted_iota(jnp.int32, (1, S), 1)
    for b in range(B):
        l = jnp.minimum(cu_ref[b + 1] - cu_ref[b], S)
        mask_ref[b:b + 1, :] = iota < l


def _mask_call(cu):
    return pl.pallas_call(
        _mask_body,
        in_specs=[pl.BlockSpec(memory_space=pltpu.SMEM)],
        out_shape=jax.ShapeDtypeStruct((B, S), jnp.bool_),
    )(cu)


def kernel(flat_encodings, cu_seqlens):
    cu = cu_seqlens.astype(jnp.int32)
    cucat = jnp.concatenate([cu[:B], cu[1:B + 1]])
    out = _sc_copy(flat_encodings, cucat)
    mask = _mask_call(cu)
    return out.reshape(B, S, D), mask


# speedup vs baseline: 46.1773x; 1.0385x over previous
import jax
import jax.numpy as jnp
from jax import lax
from jax.experimental import pallas as pl
from jax.experimental.pallas import tpu as pltpu
from jax.experimental.pallas import tpu_sc as plsc

B = 16
S = 4096
D = 512
T = 32768
NC = 2
NS = 16
NW = NC * NS
C = 32
NP = 4


def _sc_copy_body(flat_hbm, cu_hbm, out_hbm,
                  cu_v, idx0, idx1, idx2, idx3,
                  zbuf, buf0, buf1, buf2, buf3,
                  gsem0, gsem1, gsem2, gsem3,
                  ssem0, ssem1, ssem2, ssem3, zsem):
    w = lax.axis_index("s") * NC + lax.axis_index("c")
    pltpu.sync_copy(cu_hbm, cu_v.at[pl.ds(0, B + 1)])
    iota = lax.iota(jnp.int32, 16)
    zero16 = jnp.zeros((16,), jnp.float32)

    def _zb(rw, carry):
        for k in range(D // 16):
            zbuf[rw, pl.ds(k * 16, 16)] = zero16
        return carry

    lax.fori_loop(0, C, _zb, 0)

    bufs = (buf0, buf1, buf2, buf3)
    idxs = (idx0, idx1, idx2, idx3)
    gsems = (gsem0, gsem1, gsem2, gsem3)
    ssems = (ssem0, ssem1, ssem2, ssem3)

    def chunk_params(b, p):
        j = ((w + 2 * b) & 31) + 32 * p
        lo_b = cu_v[pl.ds(b, 16)][0]
        eff_b = jnp.minimum(cu_v[pl.ds(b + 1, 16)][0] - lo_b, S)
        dst = b * S + j * C
        src = lo_b + j * C
        v = jnp.clip(eff_b - j * C, 0, C)
        return src, dst, v

    def gather(p):
        return pltpu.make_async_copy(flat_hbm.at[idxs[p]], bufs[p], gsems[p])

    def scatter(dst, p):
        return pltpu.make_async_copy(
            bufs[p], out_hbm.at[pl.ds(dst, C)], ssems[p])

    def zscatter(dst):
        return pltpu.make_async_copy(zbuf, out_hbm.at[pl.ds(dst, C)], zsem)

    def _zs(b, nz):
        for p in range(NP):
            src, dst, v = chunk_params(b, p)

            @pl.when(v == 0)
            def _(dst=dst):
                zscatter(dst).start()
            nz = nz + (v == 0).astype(jnp.int32)
        return nz

    nzero = lax.fori_loop(0, B, _zs, jnp.int32(0))

    def _ring(b, pending):
        prms = [chunk_params(b, p) for p in range(NP)]
        for p in range(NP):
            src, dst, v = prms[p]

            @pl.when(v > 0)
            def _(src=src, p=p):
                @pl.when(pending[p])
                def _():
                    scatter(0, p).wait()
                for k in range(C // 16):
                    idxs[p][pl.ds(k * 16, 16)] = jnp.minimum(
                        src + k * 16 + iota, T - 1)
                gather(p).start()
        new_pending = []
        for p in range(NP):
            src, dst, v = prms[p]

            @pl.when(v > 0)
            def _(dst=dst, v=v, p=p):
                gather(p).wait()

                @pl.when(v < C)
                def _():
                    def _zr(rw, carry):
                        for k in range(D // 16):
                            bufs[p][rw, pl.ds(k * 16, 16)] = zero16
                        return carry

                    lax.fori_loop(v, C, _zr, 0)

                scatter(dst, p).start()
            new_pending.append(pending[p] | (v > 0))
        return tuple(new_pending)

    pending = lax.fori_loop(0, B, _ring, (jnp.bool_(False),) * NP)

    for p in range(NP):
        @pl.when(pending[p])
        def _(p=p):
            scatter(0, p).wait()

    def _zd(i, carry):
        zscatter(0).wait()
        return carry

    lax.fori_loop(0, nzero, _zd, 0)


_SCRATCH = (
    [pltpu.VMEM((2 * B + 1,), jnp.int32)]
    + [pltpu.VMEM((C,), jnp.int32) for _ in range(NP)]
    + [pltpu.VMEM((C, D), jnp.float32) for _ in range(NP + 1)]
    + [pltpu.SemaphoreType.DMA for _ in range(2 * NP + 1)]
)

_sc_copy = pl.kernel(
    _sc_copy_body,
    out_type=jax.ShapeDtypeStruct((B * S, D), jnp.float32),
    mesh=plsc.VectorSubcoreMesh(core_axis_name="c", subcore_axis_name="s"),
    scratch_types=_SCRATCH,
)


def _mask_body(cu_ref, mask_ref):
    iota = lax.broadcasted_iota(jnp.int32, (1, S), 1)
    for b in range(B):
        l = jnp.minimum(cu_ref[b + 1] - cu_ref[b], S)
        mask_ref[b:b + 1, :] = iota < l


def _mask_call(cu):
    return pl.pallas_call(
        _mask_body,
        in_specs=[pl.BlockSpec(memory_space=pltpu.SMEM)],
        out_shape=jax.ShapeDtypeStruct((B, S), jnp.bool_),
    )(cu)


def kernel(flat_encodings, cu_seqlens):
    cu = cu_seqlens.astype(jnp.int32)
    out = _sc_copy(flat_encodings, cu)
    mask = _mask_call(cu)
    return out.reshape(B, S, D), mask
